# Initial kernel scaffold; baseline (speedup 1.0000x reference)
import jax
import jax.numpy as jnp
from jax.experimental import pallas as pl


def kernel(edge_index, edge_type, emb, W1, root1, bias1, W2, root2, bias2):
    raise NotImplementedError("write your pallas kernel here")


# speedup vs baseline: 2.7660x; 2.7660x over previous
import functools

import jax
import jax.numpy as jnp
from jax import lax
from jax.experimental import pallas as pl
from jax.experimental.pallas import tpu as pltpu
from jax.experimental.pallas import tpu_sc as plsc

N = 10000
E = 320000
R = 16
H1 = 64
H2 = 16

NC = 2
NS = 16
NW = NC * NS
EPW = E // NW
C = 80
NSEG = N * R
SEG_PER_TILE = NSEG // NS
NP = 10240
NPT = NP // NS
ZROWS = 128

_MESH = plsc.VectorSubcoreMesh(
    core_axis_name="c", subcore_axis_name="s", num_cores=NC, num_subcores=NS
)


def _fill_vmem_1d(buf, nwords, value):
    v = jnp.full((16,), value, jnp.float32)

    def body(i, _):
        buf[pl.ds(i * 16, 16)] = v
        return 0

    lax.fori_loop(0, nwords // 16, body, 0, unroll=4)


def _zero_vmem_2d(buf, rows, do):
    z = jnp.zeros((16,), jnp.float32)

    def body(r, _):
        for q in range(do // 16):
            buf[r, pl.ds(q * 16, 16)] = z
        return 0

    lax.fori_loop(0, rows, body, 0, unroll=4)


EPS = E // NS


def _cw_body(dst_hbm, typ_hbm, w_hbm, cnt_sh, dbuf, tbuf, kbuf, ones, zbuf, wbuf):
    cid = lax.axis_index("c")
    sid = lax.axis_index("s")
    wid = cid * NS + sid

    _fill_vmem_1d(zbuf, SEG_PER_TILE, 0.0)
    _fill_vmem_1d(ones, C, 1.0)
    pltpu.sync_copy(zbuf, cnt_sh.at[pl.ds(sid * SEG_PER_TILE, SEG_PER_TILE)])
    plsc.subcore_barrier()

    def chunk(i, _):
        base = sid * EPS + i * C
        pltpu.sync_copy(dst_hbm.at[pl.ds(base, C)], dbuf)
        pltpu.sync_copy(typ_hbm.at[pl.ds(base, C)], tbuf)

        def vec(j, _):
            s = pl.ds(j * 16, 16)
            kbuf[s] = dbuf[s] * R + tbuf[s]
            return 0

        lax.fori_loop(0, C // 16, vec, 0, unroll=4)
        pltpu.sync_copy(ones, cnt_sh.at[kbuf], add=True)
        return 0

    lax.fori_loop(0, EPS // C, chunk, 0)
    plsc.subcore_barrier()

    def wchunk(i, _):
        base = wid * EPW + i * C
        pltpu.sync_copy(dst_hbm.at[pl.ds(base, C)], dbuf)
        pltpu.sync_copy(typ_hbm.at[pl.ds(base, C)], tbuf)

        def vec(j, _):
            s = pl.ds(j * 16, 16)
            kbuf[s] = dbuf[s] * R + tbuf[s]
            return 0

        lax.fori_loop(0, C // 16, vec, 0, unroll=4)
        pltpu.sync_copy(cnt_sh.at[kbuf], wbuf)

        def mkw(j, _):
            s = pl.ds(j * 16, 16)
            c = jnp.maximum(wbuf[s], 1.0)
            r = 1.0 / c
            r = r * (2.0 - c * r)
            wbuf[s] = r * (2.0 - c * r)
            return 0

        lax.fori_loop(0, C // 16, mkw, 0, unroll=4)
        pltpu.sync_copy(wbuf, w_hbm.at[pl.ds(base, C)])
        return 0

    lax.fori_loop(0, EPW // C, wchunk, 0)


def _edge_weights(dst, typ):
    return pl.kernel(
        _cw_body,
        out_type=pltpu.MemorySpace.HBM((E,), jnp.float32),
        mesh=_MESH,
        compiler_params=pltpu.CompilerParams(use_tc_tiling_on_sc=False),
        scratch_types=[
            pltpu.VMEM_SHARED((NSEG,), jnp.float32),
            pltpu.VMEM((C,), jnp.int32),
            pltpu.VMEM((C,), jnp.int32),
            pltpu.VMEM((C,), jnp.int32),
            pltpu.VMEM((C,), jnp.float32),
            pltpu.VMEM((SEG_PER_TILE,), jnp.float32),
            pltpu.VMEM((C,), jnp.float32),
        ],
    )(dst, typ)


def _agg_body(do, src_hbm, dst_hbm, typ_hbm, y_hbm, w_hbm, agg_hbm,
              agg_sh, sbuf, dbuf, tbuf, gbuf, wbuf, rbuf, zb2):
    cid = lax.axis_index("c")
    sid = lax.axis_index("s")
    wid = cid * NS + sid

    _zero_vmem_2d(zb2, ZROWS, do)

    def zero_slice(i, _):
        pltpu.sync_copy(zb2, agg_sh.at[pl.ds(sid * NPT + i * ZROWS, ZROWS)])
        return 0

    lax.fori_loop(0, NPT // ZROWS, zero_slice, 0)
    plsc.subcore_barrier()

    def chunk(i, _):
        base = wid * EPW + i * C
        pltpu.sync_copy(src_hbm.at[pl.ds(base, C)], sbuf)
        pltpu.sync_copy(typ_hbm.at[pl.ds(base, C)], tbuf)
        pltpu.sync_copy(dst_hbm.at[pl.ds(base, C)], dbuf)
        pltpu.sync_copy(w_hbm.at[pl.ds(base, C)], wbuf)

        def mkg(j, _):
            s = pl.ds(j * 16, 16)
            gbuf[s] = tbuf[s] * N + sbuf[s]
            return 0

        lax.fori_loop(0, C // 16, mkg, 0, unroll=4)
        pltpu.sync_copy(y_hbm.at[gbuf], rbuf)

        def scale(g, _):
            wv = wbuf[pl.ds(g * 16, 16)]
            for l in range(16):
                e = g * 16 + l
                w = wv[l]
                for q in range(do // 16):
                    s = pl.ds(q * 16, 16)
                    rbuf[e, s] = rbuf[e, s] * w
            return 0

        lax.fori_loop(0, C // 16, scale, 0)
        pltpu.sync_copy(rbuf, agg_sh.at[dbuf], add=True)
        return 0

    lax.fori_loop(0, EPW // C, chunk, 0)
    plsc.subcore_barrier()

    def out_slice(i, _):
        rows = pl.ds(sid * NPT + i * ZROWS, ZROWS)
        pltpu.sync_copy(agg_sh.at[rows], zb2)
        pltpu.sync_copy(zb2, agg_hbm.at[cid, rows])
        return 0

    lax.fori_loop(0, NPT // ZROWS, out_slice, 0)


def _aggregate(src, dst, typ, y_flat, w_edge, do):
    return pl.kernel(
        functools.partial(_agg_body, do),
        out_type=pltpu.MemorySpace.HBM((NC, NP, do), jnp.float32),
        mesh=_MESH,
        compiler_params=pltpu.CompilerParams(use_tc_tiling_on_sc=False),
        scratch_types=[
            pltpu.VMEM_SHARED((NP, do), jnp.float32),
            pltpu.VMEM((C,), jnp.int32),
            pltpu.VMEM((C,), jnp.int32),
            pltpu.VMEM((C,), jnp.int32),
            pltpu.VMEM((C,), jnp.int32),
            pltpu.VMEM((C,), jnp.float32),
            pltpu.VMEM((C, do), jnp.float32),
            pltpu.VMEM((ZROWS, do), jnp.float32),
        ],
    )(src, dst, typ, y_flat, w_edge)


def _mm_body(x_ref, w_ref, y_ref):
    y_ref[0] = jnp.dot(
        x_ref[...], w_ref[0],
        preferred_element_type=jnp.float32,
        precision=lax.Precision.HIGHEST,
    )


def _transform(x, w_all):
    g, din, dout = w_all.shape
    n = x.shape[0]
    return pl.pallas_call(
        _mm_body,
        grid=(g,),
        in_specs=[
            pl.BlockSpec((n, din), lambda r: (0, 0)),
            pl.BlockSpec((1, din, dout), lambda r: (r, 0, 0)),
        ],
        out_specs=pl.BlockSpec((1, n, dout), lambda r: (r, 0, 0)),
        out_shape=jax.ShapeDtypeStruct((g, n, dout), jnp.float32),
    )(x, w_all)


def _combine_relu_body(a_ref, p_ref, b_ref, h_ref):
    h_ref[...] = jnp.maximum(a_ref[0, :N] + a_ref[1, :N] + p_ref[...] + b_ref[...], 0.0)


def _combine_relu(agg, p, b):
    n, do = p.shape
    return pl.pallas_call(
        _combine_relu_body,
        out_shape=jax.ShapeDtypeStruct((n, do), jnp.float32),
    )(agg, p, b.reshape(1, do))


def _combine_sigmoid_body(a_ref, p_ref, b_ref, o_ref):
    o_ref[...] = jax.nn.sigmoid(a_ref[0, :N] + a_ref[1, :N] + p_ref[...] + b_ref[...])


def _combine_sigmoid(agg, p, b):
    n, do = p.shape
    return pl.pallas_call(
        _combine_sigmoid_body,
        out_shape=jax.ShapeDtypeStruct((n, do), jnp.float32),
    )(agg, p, b.reshape(1, do))


def kernel(edge_index, edge_type, emb, W1, root1, bias1, W2, root2, bias2):
    src = edge_index[0].astype(jnp.int32)
    dst = edge_index[1].astype(jnp.int32)
    typ = edge_type.astype(jnp.int32)

    w_edge = _edge_weights(dst, typ)

    w1_all = jnp.concatenate([W1, root1[None]], axis=0)
    y1 = _transform(emb, w1_all)
    y1_flat = y1[:R].reshape(R * N, H1)
    agg1 = _aggregate(src, dst, typ, y1_flat, w_edge, H1)
    h = _combine_relu(agg1, y1[R], bias1)

    w2_all = jnp.concatenate([W2, root2[None]], axis=0)
    y2 = _transform(h, w2_all)
    y2_flat = y2[:R].reshape(R * N, H2)
    agg2 = _aggregate(src, dst, typ, y2_flat, w_edge, H2)
    return _combine_sigmoid(agg2, y2[R], bias2)
